# baseline (speedup 1.0000x reference)
import functools

import jax
import jax.numpy as jnp
from jax import lax
from jax.experimental import pallas as pl
from jax.experimental.pallas import tpu as pltpu
from jax.experimental.pallas import tpu_sc as plsc

B = 4096
NB = 1024
N = 2048
K = 14
M = 1 << K
NH = N // 2
NQ = N // 4

NUM_WORKERS = 32
QPW = NQ // NUM_WORKERS
GRP = 2


_BN = 256
_BB = 512
_OFF = NH // _BN


_DN = (((1,), (1,)), ((), ()))


def _addr_body(wlo_a, whi_a, wlo_b, whi_b, bits_ref, out_ref):
    bits = bits_ref[...]

    def mm(w):
        return lax.dot_general(w[...], bits, _DN,
                               preferred_element_type=jnp.float32)

    a_lo = mm(wlo_a) + 128.0 * mm(whi_a)
    a_hi = mm(wlo_b) + 128.0 * mm(whi_b)
    out_ref[...] = a_lo.astype(jnp.int32) | (a_hi.astype(jnp.int32) << 16)


_addr_call = pl.pallas_call(
    _addr_body,
    grid=(NH // _BN, B // _BB),
    in_specs=[
        pl.BlockSpec((_BN, NB), lambda i, j: (i, 0)),
        pl.BlockSpec((_BN, NB), lambda i, j: (i, 0)),
        pl.BlockSpec((_BN, NB), lambda i, j: (i + _OFF, 0)),
        pl.BlockSpec((_BN, NB), lambda i, j: (i + _OFF, 0)),
        pl.BlockSpec((_BB, NB), lambda i, j: (j, 0)),
    ],
    out_specs=pl.BlockSpec((_BN, _BB), lambda i, j: (i, j)),
    out_shape=jax.ShapeDtypeStruct((NH, B), jnp.int32),
)


_TR = 128
_TC = 2048
_TOFF = NQ // _TR


def _truth_body(m0, m1, m2, m3, out_ref):
    t0 = (m0[...] == 1).astype(jnp.int32)
    t1 = (m1[...] == 1).astype(jnp.int32)
    t2 = (m2[...] == 1).astype(jnp.int32)
    t3 = (m3[...] == 1).astype(jnp.int32)
    out_ref[...] = t0 | (t1 << 8) | (t2 << 16) | (t3 << 24)


_truth_call = pl.pallas_call(
    _truth_body,
    grid=(NQ // _TR, M // _TC),
    in_specs=[
        pl.BlockSpec((_TR, _TC), lambda i, j, o=o: (i + o * _TOFF, j))
        for o in range(4)
    ],
    out_specs=pl.BlockSpec((_TR, _TC), lambda i, j: (i, j)),
    out_shape=jax.ShapeDtypeStruct((NQ, M), jnp.int32),
)


def _trans_body(w_ref, out_ref):
    k = pl.program_id(2)
    bits = lax.shift_right_logical(w_ref[...], 8 * k) & 1
    out_ref[...] = bits.T.astype(jnp.uint8)


_trans_call = pl.pallas_call(
    _trans_body,
    grid=(N // 256, (B // 4) // 128, 4),
    in_specs=[pl.BlockSpec((256, 128), lambda i, j, k: (i, j))],
    out_specs=pl.BlockSpec((128, 256), lambda i, j, k: (8 * k + j, i)),
    out_shape=jax.ShapeDtypeStruct((B, N), jnp.uint8),
)


_mesh = plsc.VectorSubcoreMesh(core_axis_name="c", subcore_axis_name="s")
BW = B // 4


NGROUPS = QPW // GRP


@functools.partial(
    pl.kernel,
    out_type=jax.ShapeDtypeStruct((N * BW,), jnp.int32),
    mesh=_mesh,
    compiler_params=pltpu.CompilerParams(needs_layout_passes=False),
    scratch_types=[
        pltpu.VMEM((GRP * M,), jnp.int32),
        pltpu.VMEM((GRP * M,), jnp.int32),
        pltpu.VMEM((2 * GRP * B,), jnp.int32),
        pltpu.VMEM((2 * GRP * B,), jnp.int32),
        pltpu.VMEM((4 * GRP * BW,), jnp.int32),
        pltpu.VMEM((4 * GRP * BW,), jnp.int32),
        pltpu.SemaphoreType.DMA,
        pltpu.SemaphoreType.DMA,
        pltpu.SemaphoreType.DMA,
        pltpu.SemaphoreType.DMA,
    ],
)
def _sc_lookup(truth_hbm, addr_hbm, out_hbm,
               rows0, rows1, addr0, addr1, outw0, outw1,
               isem0, isem1, osem0, osem1):
    wid = lax.axis_index("s") * 2 + lax.axis_index("c")
    qbase = wid * QPW
    iota4 = lax.iota(jnp.int32, 16) * 4
    rows = (rows0, rows1)
    addr = (addr0, addr1)
    outw = (outw0, outw1)
    isem = (isem0, isem1)
    osem = (osem0, osem1)

    def issue_in(g):
        p = g % 2
        q0 = qbase + g * GRP
        return (
            pltpu.async_copy(truth_hbm.at[pl.ds(q0 * M, GRP * M)],
                             rows[p], isem[p]),
            pltpu.async_copy(addr_hbm.at[pl.ds(q0 * B, GRP * B)],
                             addr[p].at[pl.ds(0, GRP * B)], isem[p]),
            pltpu.async_copy(addr_hbm.at[pl.ds((q0 + NH // 2) * B, GRP * B)],
                             addr[p].at[pl.ds(GRP * B, GRP * B)], isem[p]),
        )

    def compute(g):
        p = g % 2
        rows_v, addr_v, outw_v = rows[p], addr[p], outw[p]

        def quad(gq, _):
            ro = gq * M
            for j in range(4):
                ao = ((j & 1) * GRP + gq) * B
                sh16 = 16 * (j >> 1)
                oo = (j * GRP + gq) * BW

                def vec(v, _):
                    w = jnp.zeros((16,), jnp.int32)
                    for k in range(4):
                        aw = addr_v[pl.ds(ao + v * 16 + (B // 4) * k, 16)]
                        a = lax.shift_right_logical(aw, sh16) & 0xFFFF
                        bit = plsc.load_gather(rows_v, [ro + a])
                        bit = lax.shift_right_logical(bit, 8 * j) & 1
                        w = w | (bit << (8 * k))
                    outw_v[pl.ds(oo + v * 16, 16)] = w
                    return _

                lax.fori_loop(0, B // 64, vec, 0)
            return _

        lax.fori_loop(0, GRP, quad, 0)

    def issue_out(g):
        p = g % 2
        q0 = qbase + g * GRP
        handles = []
        for j in range(4):
            for gq in range(GRP):
                n = q0 + gq + 512 * j
                handles.append(pltpu.async_copy(
                    outw[p].at[pl.ds((j * GRP + gq) * BW, BW)],
                    out_hbm.at[pl.ds(n * BW, BW)], osem[p]))
        return handles

    in_h = {0: issue_in(0)}
    out_h = {}
    for g in range(NGROUPS):
        if g + 1 < NGROUPS:
            in_h[g + 1] = issue_in(g + 1)
        for h in in_h.pop(g):
            h.wait()
        if g - 2 >= 0:
            for h in out_h.pop(g - 2):
                h.wait()
        compute(g)
        out_h[g] = issue_out(g)
    for g in (NGROUPS - 2, NGROUPS - 1):
        for h in out_h.pop(g):
            h.wait()


def kernel(input_bits, memory, connections, binary_addresses):
    conn = connections.astype(jnp.int32)
    ba = binary_addresses.astype(jnp.int32)
    onehot = (conn[:, :, None] == jnp.arange(NB, dtype=jnp.int32)[None, None, :])
    wfull = jnp.sum(jnp.where(onehot, ba[:, :, None], 0), axis=1)
    wlo = (wfull & 127).astype(jnp.bfloat16)
    whi = (wfull >> 7).astype(jnp.bfloat16)
    bits_bf = input_bits.astype(jnp.bfloat16)

    addr_p = _addr_call(wlo, whi, wlo, whi, bits_bf)
    truth = _truth_call(memory, memory, memory, memory)

    outw = _sc_lookup(truth.reshape(NQ * M), addr_p.reshape(NH * B))

    out_u8 = _trans_call(outw.reshape(N, BW))
    return out_u8.astype(jnp.bool_)


# speedup vs baseline: 1.0989x; 1.0989x over previous
import functools

import jax
import jax.numpy as jnp
from jax import lax
from jax.experimental import pallas as pl
from jax.experimental.pallas import tpu as pltpu
from jax.experimental.pallas import tpu_sc as plsc

B = 4096
NB = 1024
N = 2048
K = 14
M = 1 << K
NH = N // 2
NQ = N // 4

NUM_WORKERS = 32
QPW = NQ // NUM_WORKERS
GRP = 2


_BN = 256
_BB = 512
_OFF = NH // _BN


_DN = (((1,), (1,)), ((), ()))


def _addr_body(wlo_a, whi_a, wlo_b, whi_b, bits_ref, out_ref):
    bits = bits_ref[...]

    def mm(w):
        return lax.dot_general(w[...], bits, _DN,
                               preferred_element_type=jnp.float32)

    a_lo = mm(wlo_a) + 128.0 * mm(whi_a)
    a_hi = mm(wlo_b) + 128.0 * mm(whi_b)
    out_ref[...] = a_lo.astype(jnp.int32) | (a_hi.astype(jnp.int32) << 16)


_addr_call = pl.pallas_call(
    _addr_body,
    grid=(NH // _BN, B // _BB),
    in_specs=[
        pl.BlockSpec((_BN, NB), lambda i, j: (i, 0)),
        pl.BlockSpec((_BN, NB), lambda i, j: (i, 0)),
        pl.BlockSpec((_BN, NB), lambda i, j: (i + _OFF, 0)),
        pl.BlockSpec((_BN, NB), lambda i, j: (i + _OFF, 0)),
        pl.BlockSpec((_BB, NB), lambda i, j: (j, 0)),
    ],
    out_specs=pl.BlockSpec((_BN, _BB), lambda i, j: (i, j)),
    out_shape=jax.ShapeDtypeStruct((NH, B), jnp.int32),
)


_TR = 128
_TC = 2048
_TOFF = NQ // _TR


def _truth_body(m0, m1, m2, m3, out_ref):
    t0 = (m0[...] == 1).astype(jnp.int32)
    t1 = (m1[...] == 1).astype(jnp.int32)
    t2 = (m2[...] == 1).astype(jnp.int32)
    t3 = (m3[...] == 1).astype(jnp.int32)
    out_ref[...] = t0 | (t1 << 8) | (t2 << 16) | (t3 << 24)


_truth_call = pl.pallas_call(
    _truth_body,
    grid=(NQ // _TR, M // _TC),
    in_specs=[
        pl.BlockSpec((_TR, _TC), lambda i, j, o=o: (i + o * _TOFF, j))
        for o in range(4)
    ],
    out_specs=pl.BlockSpec((_TR, _TC), lambda i, j: (i, j)),
    out_shape=jax.ShapeDtypeStruct((NQ, M), jnp.int32),
)


_mesh = plsc.VectorSubcoreMesh(core_axis_name="c", subcore_axis_name="s")
BW = B // 4


NGROUPS = QPW // GRP


@functools.partial(
    pl.kernel,
    out_type=jax.ShapeDtypeStruct((N * BW,), jnp.int32),
    mesh=_mesh,
    compiler_params=pltpu.CompilerParams(needs_layout_passes=False),
    scratch_types=[
        pltpu.VMEM((GRP * M,), jnp.int32),
        pltpu.VMEM((GRP * M,), jnp.int32),
        pltpu.VMEM((2 * GRP * B,), jnp.int32),
        pltpu.VMEM((2 * GRP * B,), jnp.int32),
        pltpu.VMEM((4 * GRP * BW,), jnp.int32),
        pltpu.VMEM((4 * GRP * BW,), jnp.int32),
        pltpu.SemaphoreType.DMA,
        pltpu.SemaphoreType.DMA,
        pltpu.SemaphoreType.DMA,
        pltpu.SemaphoreType.DMA,
    ],
)
def _sc_lookup(truth_hbm, addr_hbm, out_hbm,
               rows0, rows1, addr0, addr1, outw0, outw1,
               isem0, isem1, osem0, osem1):
    wid = lax.axis_index("s") * 2 + lax.axis_index("c")
    qbase = wid * QPW
    iota4 = lax.iota(jnp.int32, 16) * 4
    rows = (rows0, rows1)
    addr = (addr0, addr1)
    outw = (outw0, outw1)
    isem = (isem0, isem1)
    osem = (osem0, osem1)

    def issue_in(g):
        p = g % 2
        q0 = qbase + g * GRP
        return (
            pltpu.async_copy(truth_hbm.at[pl.ds(q0 * M, GRP * M)],
                             rows[p], isem[p]),
            pltpu.async_copy(addr_hbm.at[pl.ds(q0 * B, GRP * B)],
                             addr[p].at[pl.ds(0, GRP * B)], isem[p]),
            pltpu.async_copy(addr_hbm.at[pl.ds((q0 + NH // 2) * B, GRP * B)],
                             addr[p].at[pl.ds(GRP * B, GRP * B)], isem[p]),
        )

    def compute(g):
        p = g % 2
        rows_v, addr_v, outw_v = rows[p], addr[p], outw[p]

        def quad(gq, _):
            ro = gq * M
            for j in range(4):
                ao = ((j & 1) * GRP + gq) * B
                sh16 = 16 * (j >> 1)
                oo = (j * GRP + gq) * BW

                def vec(v, _):
                    idx0 = ao + iota4 + v * 64
                    w = jnp.zeros((16,), jnp.int32)
                    for k in range(4):
                        aw = plsc.load_gather(addr_v, [idx0 + k])
                        a = lax.shift_right_logical(aw, sh16) & 0xFFFF
                        bit = plsc.load_gather(rows_v, [ro + a])
                        bit = lax.shift_right_logical(bit, 8 * j) & 1
                        w = w | (bit << (8 * k))
                    outw_v[pl.ds(oo + v * 16, 16)] = w
                    return _

                lax.fori_loop(0, B // 64, vec, 0)
            return _

        lax.fori_loop(0, GRP, quad, 0)

    def issue_out(g):
        p = g % 2
        q0 = qbase + g * GRP
        handles = []
        for j in range(4):
            for gq in range(GRP):
                n = q0 + gq + 512 * j
                handles.append(pltpu.async_copy(
                    outw[p].at[pl.ds((j * GRP + gq) * BW, BW)],
                    out_hbm.at[pl.ds(n * BW, BW)], osem[p]))
        return handles

    in_h = {0: issue_in(0)}
    out_h = {}
    for g in range(NGROUPS):
        if g + 1 < NGROUPS:
            in_h[g + 1] = issue_in(g + 1)
        for h in in_h.pop(g):
            h.wait()
        if g - 2 >= 0:
            for h in out_h.pop(g - 2):
                h.wait()
        compute(g)
        out_h[g] = issue_out(g)
    for g in (NGROUPS - 2, NGROUPS - 1):
        for h in out_h.pop(g):
            h.wait()


def kernel(input_bits, memory, connections, binary_addresses):
    conn = connections.astype(jnp.int32)
    ba = binary_addresses.astype(jnp.int32)
    onehot = (conn[:, :, None] == jnp.arange(NB, dtype=jnp.int32)[None, None, :])
    wfull = jnp.sum(jnp.where(onehot, ba[:, :, None], 0), axis=1)
    wlo = (wfull & 127).astype(jnp.bfloat16)
    whi = (wfull >> 7).astype(jnp.bfloat16)
    bits_bf = input_bits.astype(jnp.bfloat16)

    addr_p = _addr_call(wlo, whi, wlo, whi, bits_bf)
    truth = _truth_call(memory, memory, memory, memory)

    outw = _sc_lookup(truth.reshape(NQ * M), addr_p.reshape(NH * B))

    out_u8 = lax.bitcast_convert_type(outw, jnp.uint8).reshape(N, B)
    return out_u8.T.astype(jnp.bool_)
